# baseline (speedup 1.0000x reference)
import functools

import jax
import jax.numpy as jnp
from jax.experimental import pallas as pl
from jax.experimental.pallas import tpu as pltpu
from jax.experimental.pallas import tpu_sc as plsc

_GW = 256
_NB = 400


def _sc_gather(table, idx):
    n, w = table.shape
    e = idx.shape[1]
    mesh = plsc.VectorSubcoreMesh(core_axis_name="core", subcore_axis_name="subcore")

    @pl.kernel(out_type=jax.ShapeDtypeStruct((e, w), table.dtype), mesh=mesh)
    def gather_kernel(x_hbm, i_hbm, o_hbm):
        def body(i_vmem, o_vmem):
            pltpu.sync_copy(x_hbm.at[i_vmem.at[0]], o_vmem)

        pltpu.emit_pipeline(
            body,
            grid=(e // _GW,),
            in_specs=[pl.BlockSpec((1, _GW), index_map=lambda i: (0, i))],
            out_specs=[pl.BlockSpec((_GW, w), index_map=lambda i: (i, 0))],
            core_axis_name=("core", "subcore"),
            dimension_semantics=(pltpu.PARALLEL,),
        )(i_hbm, o_hbm)

    return gather_kernel(table, idx)


def _ln(x, g, b, eps=1e-5):
    mu = jnp.mean(x, axis=-1, keepdims=True)
    var = jnp.mean((x - mu) ** 2, axis=-1, keepdims=True)
    return (x - mu) * jax.lax.rsqrt(var + eps) * g + b


def _tc_body(nb, deg, kh, gathered_ref, kin2_ref, wk1d_ref, bk1d_ref, wk2_ref,
             bk2_ref, wli_ref, bli_ref, ln1g_ref, ln1b_ref, w1_ref, b1_ref,
             w2_ref, b2_ref, ln2g_ref, ln2b_ref, out_ref):
    f32 = jnp.float32
    bf16 = jnp.bfloat16
    eb2 = kin2_ref.shape[0]
    q = jnp.dot(kin2_ref[...], wk1d_ref[...],
                preferred_element_type=f32) + bk1d_ref[...]
    hg = jax.nn.gelu(q.astype(bf16))
    kern_a = jnp.dot(hg[:, :kh], wk2_ref[...],
                     preferred_element_type=f32) + bk2_ref[...]
    kern_b = jnp.dot(hg[:, kh:], wk2_ref[...],
                     preferred_element_type=f32) + bk2_ref[...]
    c = kern_a.shape[-1]
    msg_a = gathered_ref[0:eb2, :] * kern_a
    msg_b = gathered_ref[eb2:, :] * kern_b
    s_a = jnp.sum(msg_a.reshape(nb // 2, deg, c), axis=1)
    s_b = jnp.sum(msg_b.reshape(nb // 2, deg, c), axis=1)
    s = jnp.concatenate([s_a, s_b], axis=0) * (1.0 / deg)
    xt = jnp.dot(s.astype(bf16), wli_ref[...], preferred_element_type=f32) \
        + bli_ref[...]
    xt = _ln(xt, ln1g_ref[...], ln1b_ref[...])
    hh = jax.nn.gelu(jnp.dot(xt.astype(bf16), w1_ref[...],
                             preferred_element_type=f32) + b1_ref[...])
    hh = jnp.dot(hh.astype(bf16), w2_ref[...],
                 preferred_element_type=f32) + b2_ref[...]
    out_ref[...] = _ln(hh, ln2g_ref[...], ln2b_ref[...])


def kernel(x, in_points, out_points, neighbors_index, neighbors_row_splits,
           Wk1, bk1, Wk2, bk2, Wli, bli, ln1_g, ln1_b, W1, b1, W2, b2,
           ln2_g, ln2_b):
    xs = x[0]
    n, c = xs.shape
    e = neighbors_index.shape[1]
    deg = e // n
    bf16 = jnp.bfloat16
    f32 = jnp.float32

    nchunks = 5
    ce = e // nchunks
    gchunks = [
        _sc_gather(xs,
                   jax.lax.dynamic_slice_in_dim(neighbors_index, k * ce, ce, 1))
        for k in range(nchunks)
    ]

    kin = jnp.concatenate(
        [in_points[0], out_points[0],
         jnp.zeros((e, 2), dtype=in_points.dtype)], axis=-1).astype(bf16)
    nb = _NB
    eb = nb * deg
    nblk = n // nb
    kin2 = (kin.reshape(nblk, 2, eb // 2, 8)
            .transpose(0, 2, 1, 3)
            .reshape(e // 2, 16))

    wk1p = jnp.concatenate(
        [Wk1, jnp.zeros((2, Wk1.shape[1]), dtype=Wk1.dtype)], axis=0)
    z = jnp.zeros_like(wk1p)
    wk1d = jnp.concatenate(
        [jnp.concatenate([wk1p, z], axis=1),
         jnp.concatenate([z, wk1p], axis=1)], axis=0).astype(bf16)
    bk1d = jnp.concatenate([bk1, bk1]).reshape(1, -1)

    kh = Wk1.shape[1]
    mh = W1.shape[1]

    row = lambda v: v.reshape(1, -1)
    full = lambda shp: pl.BlockSpec(shp, lambda i: (0, 0))

    cn = n // nchunks
    tc = pl.pallas_call(
        functools.partial(_tc_body, nb, deg, kh),
        grid=(cn // nb,),
        in_specs=[
            pl.BlockSpec((eb, c), lambda i: (i, 0)),
            pl.BlockSpec((eb // 2, 16), lambda i: (i, 0)),
            full((16, 2 * kh)), full((1, 2 * kh)),
            full((kh, c)), full((1, c)),
            full((c, c)), full((1, c)),
            full((1, c)), full((1, c)),
            full((c, mh)), full((1, mh)),
            full((mh, c)), full((1, c)),
            full((1, c)), full((1, c)),
        ],
        out_specs=pl.BlockSpec((nb, c), lambda i: (i, 0)),
        out_shape=jax.ShapeDtypeStruct((cn, c), f32),
    )

    outs = [
        tc(gchunks[k],
           jax.lax.dynamic_slice_in_dim(kin2, k * (ce // 2), ce // 2, 0),
           wk1d, bk1d, Wk2.astype(bf16), row(bk2),
           Wli.astype(bf16), row(bli), row(ln1_g), row(ln1_b),
           W1.astype(bf16), row(b1), W2.astype(bf16), row(b2),
           row(ln2_g), row(ln2_b))
        for k in range(nchunks)
    ]
    return jnp.concatenate(outs, axis=0)[None]


# speedup vs baseline: 1.1977x; 1.1977x over previous
import functools

import jax
import jax.numpy as jnp
from jax.experimental import pallas as pl
from jax.experimental.pallas import tpu as pltpu
from jax.experimental.pallas import tpu_sc as plsc

_GW = 256
_NB = 400


def _sc_gather(table, idx):
    n, w = table.shape
    e = idx.shape[1]
    mesh = plsc.VectorSubcoreMesh(core_axis_name="core", subcore_axis_name="subcore")

    @pl.kernel(out_type=jax.ShapeDtypeStruct((e, w), table.dtype), mesh=mesh)
    def gather_kernel(x_hbm, i_hbm, o_hbm):
        def body(i_vmem, o_vmem):
            pltpu.sync_copy(x_hbm.at[i_vmem.at[0]], o_vmem)

        pltpu.emit_pipeline(
            body,
            grid=(e // _GW,),
            in_specs=[pl.BlockSpec((1, _GW), index_map=lambda i: (0, i))],
            out_specs=[pl.BlockSpec((_GW, w), index_map=lambda i: (i, 0))],
            core_axis_name=("core", "subcore"),
            dimension_semantics=(pltpu.PARALLEL,),
        )(i_hbm, o_hbm)

    return gather_kernel(table, idx)


def _ln(x, g, b, eps=1e-5):
    mu = jnp.mean(x, axis=-1, keepdims=True)
    var = jnp.mean((x - mu) ** 2, axis=-1, keepdims=True)
    return (x - mu) * jax.lax.rsqrt(var + eps) * g + b


def _tc_body(nb, deg, kh, gathered_ref, kin2_ref, wk1d_ref, bk1d_ref, wk2_ref,
             bk2_ref, wli_ref, bli_ref, ln1g_ref, ln1b_ref, w1_ref, b1_ref,
             w2_ref, b2_ref, ln2g_ref, ln2b_ref, out_ref):
    f32 = jnp.float32
    bf16 = jnp.bfloat16
    eb2 = kin2_ref.shape[0]
    q = jnp.dot(kin2_ref[...], wk1d_ref[...],
                preferred_element_type=f32) + bk1d_ref[...]
    hg = jax.nn.gelu(q.astype(bf16))
    kern_a = jnp.dot(hg[:, :kh], wk2_ref[...],
                     preferred_element_type=f32) + bk2_ref[...]
    kern_b = jnp.dot(hg[:, kh:], wk2_ref[...],
                     preferred_element_type=f32) + bk2_ref[...]
    c = kern_a.shape[-1]
    msg_a = gathered_ref[0:eb2, :] * kern_a
    msg_b = gathered_ref[eb2:, :] * kern_b
    s_a = jnp.sum(msg_a.reshape(nb // 2, deg, c), axis=1)
    s_b = jnp.sum(msg_b.reshape(nb // 2, deg, c), axis=1)
    s = jnp.concatenate([s_a, s_b], axis=0) * (1.0 / deg)
    xt = jnp.dot(s.astype(bf16), wli_ref[...], preferred_element_type=f32) \
        + bli_ref[...]
    xt = _ln(xt, ln1g_ref[...], ln1b_ref[...])
    hh = jax.nn.gelu(jnp.dot(xt.astype(bf16), w1_ref[...],
                             preferred_element_type=f32) + b1_ref[...])
    hh = jnp.dot(hh.astype(bf16), w2_ref[...],
                 preferred_element_type=f32) + b2_ref[...]
    out_ref[...] = _ln(hh, ln2g_ref[...], ln2b_ref[...])


def kernel(x, in_points, out_points, neighbors_index, neighbors_row_splits,
           Wk1, bk1, Wk2, bk2, Wli, bli, ln1_g, ln1_b, W1, b1, W2, b2,
           ln2_g, ln2_b):
    xs = x[0]
    n, c = xs.shape
    e = neighbors_index.shape[1]
    deg = e // n
    bf16 = jnp.bfloat16
    f32 = jnp.float32

    gathered = _sc_gather(xs, neighbors_index)

    kin = jnp.concatenate(
        [in_points[0], out_points[0],
         jnp.zeros((e, 2), dtype=in_points.dtype)], axis=-1).astype(bf16)
    nb = _NB
    eb = nb * deg
    nblk = n // nb
    kin2 = (kin.reshape(nblk, 2, eb // 2, 8)
            .transpose(0, 2, 1, 3)
            .reshape(e // 2, 16))

    wk1p = jnp.concatenate(
        [Wk1, jnp.zeros((2, Wk1.shape[1]), dtype=Wk1.dtype)], axis=0)
    z = jnp.zeros_like(wk1p)
    wk1d = jnp.concatenate(
        [jnp.concatenate([wk1p, z], axis=1),
         jnp.concatenate([z, wk1p], axis=1)], axis=0).astype(bf16)
    bk1d = jnp.concatenate([bk1, bk1]).reshape(1, -1)

    kh = Wk1.shape[1]
    mh = W1.shape[1]

    row = lambda v: v.reshape(1, -1)
    full = lambda shp: pl.BlockSpec(shp, lambda i: (0, 0))

    tc = pl.pallas_call(
        functools.partial(_tc_body, nb, deg, kh),
        grid=(nblk,),
        in_specs=[
            pl.BlockSpec((eb, c), lambda i: (i, 0)),
            pl.BlockSpec((eb // 2, 16), lambda i: (i, 0)),
            full((16, 2 * kh)), full((1, 2 * kh)),
            full((kh, c)), full((1, c)),
            full((c, c)), full((1, c)),
            full((1, c)), full((1, c)),
            full((c, mh)), full((1, mh)),
            full((mh, c)), full((1, c)),
            full((1, c)), full((1, c)),
        ],
        out_specs=pl.BlockSpec((nb, c), lambda i: (i, 0)),
        out_shape=jax.ShapeDtypeStruct((n, c), f32),
    )

    out = tc(gathered, kin2, wk1d, bk1d, Wk2.astype(bf16), row(bk2),
             Wli.astype(bf16), row(bli), row(ln1_g), row(ln1_b),
             W1.astype(bf16), row(b1), W2.astype(bf16), row(b2),
             row(ln2_g), row(ln2_b))
    return out[None]
